# Initial kernel scaffold; baseline (speedup 1.0000x reference)
import functools

import jax
import jax.numpy as jnp
from jax import lax
from jax.experimental import pallas as pl
from jax.experimental.pallas import tpu as pltpu
from jax.experimental.pallas import tpu_sc as plsc

D = 64
G = 128
KF = 8
NC = 2
NS = 16
NW = NC * NS


def _embed_sc(x_rows, table):
    rows = x_rows.shape[0]
    per_w = rows // NW
    steps = per_w // KF

    mesh = plsc.VectorSubcoreMesh(core_axis_name="c", subcore_axis_name="s")

    @functools.partial(
        pl.kernel,
        out_type=jax.ShapeDtypeStruct((rows, G, D), jnp.float32),
        mesh=mesh,
        scratch_types=[
            pltpu.VMEM((KF, G), jnp.int32),
            pltpu.VMEM((KF, G, D), jnp.float32),
            pltpu.SemaphoreType.DMA,
        ],
    )
    def k(idx_hbm, tab_hbm, out_hbm, idx_v, rows_v, sem):
        wid = lax.axis_index("s") * NC + lax.axis_index("c")
        base = wid * per_w

        def step(i):
            r0 = base + i * KF
            pltpu.sync_copy(idx_hbm.at[pl.ds(r0, KF)], idx_v)
            copies = [
                pltpu.async_copy(tab_hbm.at[idx_v.at[j]], rows_v.at[j], sem)
                for j in range(KF)
            ]
            for c in copies:
                c.wait()
            pltpu.sync_copy(rows_v, out_hbm.at[pl.ds(r0, KF)])

        pl.loop(0, steps)(step)

    return k(x_rows, table)


def kernel(x, table):
    b, s = x.shape
    x_rows = x.reshape(b * s // G, G).astype(jnp.int32)
    out = _embed_sc(x_rows, table)
    return out.reshape(b, s, D)


# speedup vs baseline: 4.1317x; 4.1317x over previous
import functools

import jax
import jax.numpy as jnp
from jax import lax
from jax.experimental import pallas as pl
from jax.experimental.pallas import tpu as pltpu
from jax.experimental.pallas import tpu_sc as plsc

D = 64
G = 128
KF = 8
NC = 2
NS = 16
NW = NC * NS


def _embed_sc(x_rows, table):
    rows = x_rows.shape[0]
    per_w = rows // NW
    steps = per_w // KF

    mesh = plsc.VectorSubcoreMesh(core_axis_name="c", subcore_axis_name="s")

    @functools.partial(
        pl.kernel,
        out_type=jax.ShapeDtypeStruct((rows, G, D), jnp.float32),
        mesh=mesh,
        scratch_types=[
            pltpu.VMEM((KF, G), jnp.int32),
            pltpu.VMEM((KF, G, D), jnp.float32),
            pltpu.SemaphoreType.DMA,
        ],
        compiler_params=pltpu.CompilerParams(use_tc_tiling_on_sc=False),
    )
    def k(idx_hbm, tab_hbm, out_hbm, idx_v, rows_v, sem):
        wid = lax.axis_index("s") * NC + lax.axis_index("c")
        base = wid * per_w

        def step(i):
            r0 = base + i * KF
            pltpu.sync_copy(idx_hbm.at[pl.ds(r0, KF)], idx_v)
            copies = [
                pltpu.async_copy(tab_hbm.at[idx_v.at[j]], rows_v.at[j], sem)
                for j in range(KF)
            ]
            for c in copies:
                c.wait()
            pltpu.sync_copy(rows_v, out_hbm.at[pl.ds(r0, KF)])

        pl.loop(0, steps)(step)

    return k(x_rows, table)


def kernel(x, table):
    b, s = x.shape
    x_rows = x.reshape(b * s // G, G).astype(jnp.int32)
    out = _embed_sc(x_rows, table)
    return out.reshape(b, s, D)


# speedup vs baseline: 4.2577x; 1.0305x over previous
import functools

import jax
import jax.numpy as jnp
from jax import lax
from jax.experimental import pallas as pl
from jax.experimental.pallas import tpu as pltpu
from jax.experimental.pallas import tpu_sc as plsc

D = 64
G = 128
KB = 4
NB = 2
NC = 2
NS = 16
NW = NC * NS


def _embed_sc(x_rows, table):
    rows = x_rows.shape[0]
    per_w = rows // NW
    nsteps = per_w // KB

    mesh = plsc.VectorSubcoreMesh(core_axis_name="c", subcore_axis_name="s")

    @functools.partial(
        pl.kernel,
        out_type=jax.ShapeDtypeStruct((rows, G, D), jnp.float32),
        mesh=mesh,
        scratch_types=[
            pltpu.VMEM((per_w, G), jnp.int32),
            pltpu.VMEM((NB, KB, G, D), jnp.float32),
            pltpu.SemaphoreType.DMA,
            pltpu.SemaphoreType.DMA,
        ],
        compiler_params=pltpu.CompilerParams(use_tc_tiling_on_sc=False),
    )
    def k(idx_hbm, tab_hbm, out_hbm, idx_v, rows_v, sem0, sem1):
        sems = [sem0, sem1]
        wid = lax.axis_index("s") * NC + lax.axis_index("c")
        base = wid * per_w
        pltpu.sync_copy(idx_hbm.at[pl.ds(base, per_w)], idx_v)

        def fire(i, b):
            for j in range(KB):
                pltpu.async_copy(
                    tab_hbm.at[idx_v.at[i * KB + j]],
                    rows_v.at[b].at[j],
                    sems[b],
                )

        def drain_write(i, b):
            for j in range(KB):
                pltpu.make_async_copy(
                    tab_hbm.at[idx_v.at[i * KB + j]],
                    rows_v.at[b].at[j],
                    sems[b],
                ).wait()
            pltpu.sync_copy(rows_v.at[b], out_hbm.at[pl.ds(base + i * KB, KB)])

        fire(0, 0)

        def body(i):
            for b in range(NB):
                cur = i + b
                nxt = cur + 1

                @pl.when(nxt < nsteps)
                def _():
                    fire(nxt, (b + 1) % NB)

                drain_write(cur, b)

        pl.loop(0, nsteps, step=NB)(body)

    return k(x_rows, table)


def kernel(x, table):
    b, s = x.shape
    x_rows = x.reshape(b * s // G, G).astype(jnp.int32)
    out = _embed_sc(x_rows, table)
    return out.reshape(b, s, D)


# speedup vs baseline: 4.2599x; 1.0005x over previous
import functools

import jax
import jax.numpy as jnp
from jax import lax
from jax.experimental import pallas as pl
from jax.experimental.pallas import tpu as pltpu
from jax.experimental.pallas import tpu_sc as plsc

D = 64
GS = (128, 72)
RB = 2
NB = 2
NC = 2
NS = 16
NW = NC * NS


def _embed_sc(x, table):
    B, S = x.shape
    b_per_w = B // NW
    nsteps = b_per_w // RB

    mesh = plsc.VectorSubcoreMesh(core_axis_name="c", subcore_axis_name="s")

    @functools.partial(
        pl.kernel,
        out_type=jax.ShapeDtypeStruct((B, S, D), jnp.float32),
        mesh=mesh,
        scratch_types=[
            pltpu.VMEM((b_per_w, S), jnp.int32),
            pltpu.VMEM((NB, RB, S, D), jnp.float32),
            pltpu.SemaphoreType.DMA,
            pltpu.SemaphoreType.DMA,
        ],
        compiler_params=pltpu.CompilerParams(use_tc_tiling_on_sc=False),
    )
    def k(idx_hbm, tab_hbm, out_hbm, idx_v, rows_v, sem0, sem1):
        sems = [sem0, sem1]
        wid = lax.axis_index("s") * NC + lax.axis_index("c")
        base = wid * b_per_w
        pltpu.sync_copy(idx_hbm.at[pl.ds(base, b_per_w)], idx_v)

        def fire(i, b):
            for r in range(RB):
                off = 0
                for g in GS:
                    pltpu.async_copy(
                        tab_hbm.at[idx_v.at[i * RB + r].at[pl.ds(off, g)]],
                        rows_v.at[b].at[r].at[pl.ds(off, g)],
                        sems[b],
                    )
                    off += g

        def drain_write(i, b):
            for r in range(RB):
                off = 0
                for g in GS:
                    pltpu.make_async_copy(
                        tab_hbm.at[idx_v.at[i * RB + r].at[pl.ds(off, g)]],
                        rows_v.at[b].at[r].at[pl.ds(off, g)],
                        sems[b],
                    ).wait()
                    off += g
            pltpu.sync_copy(rows_v.at[b], out_hbm.at[pl.ds(base + i * RB, RB)])

        fire(0, 0)

        def body(i):
            for b in range(NB):
                cur = i + b
                nxt = cur + 1

                @pl.when(nxt < nsteps)
                def _():
                    fire(nxt, (b + 1) % NB)

                drain_write(cur, b)

        pl.loop(0, nsteps, step=NB)(body)

    return k(x, table)


def kernel(x, table):
    return _embed_sc(x.astype(jnp.int32), table)
